# baseline (speedup 1.0000x reference)
import jax
import jax.numpy as jnp
from jax import lax
from jax.experimental import pallas as pl
from jax.experimental.pallas import tpu as pltpu
from jax.experimental.pallas import tpu_sc as plsc

B = 64
N = 512
F = 9
H = 128
NC = 2
NS = 16
NW = NC * NS
NODES = B * N
NODES_PER_W = NODES // NW
C = 16
DEPTH = 4
CHUNKS = NODES_PER_W // C
RPC = C * F
XPW = NODES_PER_W * F
NPAD = 520


def _sc_body(x_hbm, ind_hbm, outd_hbm, atom_hbm, inemb_hbm, outemb_hbm,
             gt_hbm, out_hbm,
             xidx_v, inidx_v, outidx_v, rows_b, drows_b, obuf0, obuf1,
             gt_v, sems, semw0, semw1):
    wid = lax.axis_index("s") * NC + lax.axis_index("c")
    b0 = wid * 2
    nbase = pl.multiple_of(wid * NODES_PER_W, 8)

    pltpu.sync_copy(x_hbm.at[pl.ds(pl.multiple_of(wid * XPW, 8), XPW)],
                    xidx_v)
    pltpu.sync_copy(ind_hbm.at[pl.ds(nbase, NODES_PER_W)], inidx_v)
    pltpu.sync_copy(outd_hbm.at[pl.ds(nbase, NODES_PER_W)], outidx_v)

    pltpu.sync_copy(gt_hbm.at[pl.ds(0, 1)], gt_v)
    pltpu.sync_copy(gt_v, out_hbm.at[b0, pl.ds(0, 1)])
    pltpu.sync_copy(gt_v, out_hbm.at[b0 + 1, pl.ds(0, 1)])

    def gather_ops(cc, rows, drows, sem):
        base = pl.multiple_of(cc * RPC, 8)
        dbase = pl.multiple_of(cc * C, 8)
        ops = []
        for j in range(0, RPC, 128):
            w = min(128, RPC - j)
            ops.append(pltpu.make_async_copy(
                atom_hbm.at[xidx_v.at[pl.ds(base + j, w)]],
                rows.at[pl.ds(j, w)], sem))
        ops.append(pltpu.make_async_copy(
            inemb_hbm.at[inidx_v.at[pl.ds(dbase, C)]], drows.at[0], sem))
        ops.append(pltpu.make_async_copy(
            outemb_hbm.at[outidx_v.at[pl.ds(dbase, C)]], drows.at[1], sem))
        return ops

    def fire(cc, rows, drows, sem):
        for op in gather_ops(cc, rows, drows, sem):
            op.start()

    def drain(cc, rows, drows, sem):
        for op in gather_ops(cc, rows, drows, sem):
            op.wait()

    def out_op(cc, obuf, sem):
        g0 = nbase + cc * C
        b = g0 // N
        n0 = g0 % N
        return pltpu.make_async_copy(obuf, out_hbm.at[b, pl.ds(1 + n0, C)],
                                     sem)

    def compute_and_store(cc, rows, drows, obuf, semw, first):
        @pl.when(jnp.logical_not(first))
        def _():
            out_op(cc, obuf, semw).wait()

        @plsc.parallel_loop(0, C, 1, unroll=2)
        def node_body(n):
            r0 = n * F
            for k in range(H // 16):
                sl = pl.ds(k * 16, 16)
                acc = drows[0, n, sl] + drows[1, n, sl]
                for f in range(F):
                    acc = acc + rows[r0 + f, sl]
                obuf[n, sl] = acc

        out_op(cc, obuf, semw).start()

    for p in range(DEPTH - 1):
        fire(p, rows_b.at[p], drows_b.at[p], sems.at[p])

    def loop_body(g, carry):
        for p in range(DEPTH):
            c = g * DEPTH + p
            fire_c = c + DEPTH - 1
            @pl.when(fire_c < CHUNKS)
            def _():
                q = (DEPTH - 1 + p) % DEPTH
                fire(fire_c, rows_b.at[q], drows_b.at[q], sems.at[q])
            drain(c, rows_b.at[p], drows_b.at[p], sems.at[p])
            obuf = obuf0 if p % 2 == 0 else obuf1
            semw = semw0 if p % 2 == 0 else semw1
            compute_and_store(c, rows_b.at[p], drows_b.at[p], obuf, semw,
                              g * DEPTH + p < 2)
        return carry

    lax.fori_loop(0, CHUNKS // DEPTH, loop_body, 0)
    out_op(CHUNKS - 2, obuf0, semw0).wait()
    out_op(CHUNKS - 1, obuf1, semw1).wait()


@jax.jit
def _run(x_flat, ind_flat, outd_flat, atom_t, in_deg_emb, out_deg_emb,
         gt8):
    mesh = plsc.VectorSubcoreMesh(core_axis_name="c", subcore_axis_name="s")
    out = pl.kernel(
        _sc_body,
        out_type=jax.ShapeDtypeStruct((B, NPAD, H), jnp.float32),
        mesh=mesh,
        compiler_params=pltpu.CompilerParams(use_tc_tiling_on_sc=False),
        scratch_types=[
            pltpu.VMEM((XPW,), jnp.int32),
            pltpu.VMEM((NODES_PER_W,), jnp.int32),
            pltpu.VMEM((NODES_PER_W,), jnp.int32),
            pltpu.VMEM((DEPTH, RPC, H), jnp.float32),
            pltpu.VMEM((DEPTH, 2, C, H), jnp.float32),
            pltpu.VMEM((C, H), jnp.float32),
            pltpu.VMEM((C, H), jnp.float32),
            pltpu.VMEM((1, H), jnp.float32),
            pltpu.SemaphoreType.DMA((DEPTH,)),
            pltpu.SemaphoreType.DMA,
            pltpu.SemaphoreType.DMA,
        ],
    )(x_flat, ind_flat, outd_flat, atom_t, in_deg_emb, out_deg_emb, gt8)
    return out[:, :N + 1, :]


def kernel(x, in_degree, out_degree, atom_emb, in_deg_emb, out_deg_emb,
           graph_token):
    x_flat = jnp.minimum(x.astype(jnp.int32).reshape(-1), 100000)
    ind_flat = jnp.minimum(in_degree.astype(jnp.int32).reshape(-1), 511)
    outd_flat = jnp.minimum(out_degree.astype(jnp.int32).reshape(-1), 511)
    gt8 = jnp.tile(graph_token, (8, 1))
    return _run(x_flat, ind_flat, outd_flat, atom_emb, in_deg_emb,
                out_deg_emb, gt8)


# speedup vs baseline: 1.1025x; 1.1025x over previous
import jax
import jax.numpy as jnp
from jax import lax
from jax.experimental import pallas as pl
from jax.experimental.pallas import tpu as pltpu
from jax.experimental.pallas import tpu_sc as plsc

B = 64
N = 512
F = 9
H = 128
NC = 2
NS = 16
NW = NC * NS
NODES = B * N
NODES_PER_W = NODES // NW
C = 32
CHUNKS = NODES_PER_W // C
RPC = C * F
XPW = NODES_PER_W * F
NPAD = 520


def _sc_body(x_hbm, ind_hbm, outd_hbm, atom_hbm, inemb_hbm, outemb_hbm,
             gt_hbm, out_hbm,
             xidx_v, inidx_v, outidx_v, rows0, rows1, drows0, drows1,
             obuf0, obuf1, gt_v, sem0, sem1, semw0, semw1):
    wid = lax.axis_index("s") * NC + lax.axis_index("c")
    b0 = wid * 2
    nbase = pl.multiple_of(wid * NODES_PER_W, 8)

    pltpu.sync_copy(x_hbm.at[pl.ds(pl.multiple_of(wid * XPW, 8), XPW)],
                    xidx_v)
    pltpu.sync_copy(ind_hbm.at[pl.ds(nbase, NODES_PER_W)], inidx_v)
    pltpu.sync_copy(outd_hbm.at[pl.ds(nbase, NODES_PER_W)], outidx_v)

    pltpu.sync_copy(gt_hbm.at[pl.ds(0, 1)], gt_v)
    pltpu.sync_copy(gt_v, out_hbm.at[b0, pl.ds(0, 1)])
    pltpu.sync_copy(gt_v, out_hbm.at[b0 + 1, pl.ds(0, 1)])

    def gather_ops(cc, rows, drows, sem):
        base = pl.multiple_of(cc * RPC, 8)
        dbase = pl.multiple_of(cc * C, 8)
        ops = []
        for j in range(0, RPC, 128):
            w = min(128, RPC - j)
            ops.append(pltpu.make_async_copy(
                atom_hbm.at[xidx_v.at[pl.ds(base + j, w)]],
                rows.at[pl.ds(j, w)], sem))
        ops.append(pltpu.make_async_copy(
            inemb_hbm.at[inidx_v.at[pl.ds(dbase, C)]], drows.at[0], sem))
        ops.append(pltpu.make_async_copy(
            outemb_hbm.at[outidx_v.at[pl.ds(dbase, C)]], drows.at[1], sem))
        return ops

    def fire(cc, rows, drows, sem):
        for op in gather_ops(cc, rows, drows, sem):
            op.start()

    def drain(cc, rows, drows, sem):
        for op in gather_ops(cc, rows, drows, sem):
            op.wait()

    def out_op(cc, obuf, sem):
        g0 = nbase + cc * C
        b = g0 // N
        n0 = g0 % N
        return pltpu.make_async_copy(obuf, out_hbm.at[b, pl.ds(1 + n0, C)],
                                     sem)

    def compute_and_store(cc, rows, drows, obuf, semw, first):
        @pl.when(jnp.logical_not(first))
        def _():
            out_op(cc, obuf, semw).wait()

        @plsc.parallel_loop(0, C, 1, unroll=2)
        def node_body(n):
            r0 = n * F
            for k in range(H // 16):
                sl = pl.ds(k * 16, 16)
                acc = drows[0, n, sl] + drows[1, n, sl]
                for f in range(F):
                    acc = acc + rows[r0 + f, sl]
                obuf[n, sl] = acc

        out_op(cc, obuf, semw).start()

    fire(0, rows0, drows0, sem0)

    def loop_body(g, carry):
        c0 = g * 2
        c1 = c0 + 1
        fire(c1, rows1, drows1, sem1)
        drain(c0, rows0, drows0, sem0)
        compute_and_store(c0, rows0, drows0, obuf0, semw0, g == 0)

        @pl.when(g < CHUNKS // 2 - 1)
        def _():
            fire(c0 + 2, rows0, drows0, sem0)
        drain(c1, rows1, drows1, sem1)
        compute_and_store(c1, rows1, drows1, obuf1, semw1, g == 0)
        return carry

    lax.fori_loop(0, CHUNKS // 2, loop_body, 0)
    out_op(CHUNKS - 2, obuf0, semw0).wait()
    out_op(CHUNKS - 1, obuf1, semw1).wait()


@jax.jit
def _run(x_flat, ind_flat, outd_flat, atom_t, in_deg_emb, out_deg_emb,
         gt8):
    mesh = plsc.VectorSubcoreMesh(core_axis_name="c", subcore_axis_name="s")
    out = pl.kernel(
        _sc_body,
        out_type=jax.ShapeDtypeStruct((B, NPAD, H), jnp.float32),
        mesh=mesh,
        compiler_params=pltpu.CompilerParams(use_tc_tiling_on_sc=False),
        scratch_types=[
            pltpu.VMEM((XPW,), jnp.int32),
            pltpu.VMEM((NODES_PER_W,), jnp.int32),
            pltpu.VMEM((NODES_PER_W,), jnp.int32),
            pltpu.VMEM((RPC, H), jnp.float32),
            pltpu.VMEM((RPC, H), jnp.float32),
            pltpu.VMEM((2, C, H), jnp.float32),
            pltpu.VMEM((2, C, H), jnp.float32),
            pltpu.VMEM((C, H), jnp.float32),
            pltpu.VMEM((C, H), jnp.float32),
            pltpu.VMEM((1, H), jnp.float32),
            pltpu.SemaphoreType.DMA,
            pltpu.SemaphoreType.DMA,
            pltpu.SemaphoreType.DMA,
            pltpu.SemaphoreType.DMA,
        ],
    )(x_flat, ind_flat, outd_flat, atom_t, in_deg_emb, out_deg_emb, gt8)
    return out[:, :N + 1, :]


def kernel(x, in_degree, out_degree, atom_emb, in_deg_emb, out_deg_emb,
           graph_token):
    x_flat = jnp.minimum(x.astype(jnp.float32),
                         100000.0).astype(jnp.int32).reshape(-1)
    ind_flat = jnp.minimum(in_degree.astype(jnp.int32).reshape(-1), 511)
    outd_flat = jnp.minimum(out_degree.astype(jnp.int32).reshape(-1), 511)
    gt8 = jnp.tile(graph_token, (8, 1))
    return _run(x_flat, ind_flat, outd_flat, atom_emb, in_deg_emb,
                out_deg_emb, gt8)


# speedup vs baseline: 1.2873x; 1.1676x over previous
import jax
import jax.numpy as jnp
from jax import lax
from jax.experimental import pallas as pl
from jax.experimental.pallas import tpu as pltpu
from jax.experimental.pallas import tpu_sc as plsc

B = 64
N = 512
F = 9
H = 128
NC = 2
NS = 16
NW = NC * NS
NODES = B * N
NODES_PER_W = NODES // NW
C = 32
CHUNKS = NODES_PER_W // C
RPC = C * F
XPW = NODES_PER_W * F


def _sc_body(x_hbm, ind_hbm, outd_hbm, atom_hbm, inemb_hbm, outemb_hbm,
             gt_hbm, out_hbm,
             xidx_v, inidx_v, outidx_v, rows0, rows1, drows0, drows1,
             obuf0, obuf1, gt_v, sem0, sem1, semw0, semw1):
    wid = lax.axis_index("s") * NC + lax.axis_index("c")
    b0 = wid * 2
    nbase = pl.multiple_of(wid * NODES_PER_W, 8)

    pltpu.sync_copy(x_hbm.at[pl.ds(pl.multiple_of(wid * XPW, 8), XPW)],
                    xidx_v)
    pltpu.sync_copy(ind_hbm.at[pl.ds(nbase, NODES_PER_W)], inidx_v)
    pltpu.sync_copy(outd_hbm.at[pl.ds(nbase, NODES_PER_W)], outidx_v)

    pltpu.sync_copy(gt_hbm.at[pl.ds(0, 2)], gt_v)
    pltpu.sync_copy(gt_v, out_hbm.at[0, pl.ds(b0, 2)])

    def gather_ops(cc, rows, drows, sem):
        base = pl.multiple_of(cc * RPC, 8)
        dbase = pl.multiple_of(cc * C, 8)
        ops = []
        for j in range(0, RPC, 128):
            w = min(128, RPC - j)
            ops.append(pltpu.make_async_copy(
                atom_hbm.at[xidx_v.at[pl.ds(base + j, w)]],
                rows.at[pl.ds(j, w)], sem))
        ops.append(pltpu.make_async_copy(
            inemb_hbm.at[inidx_v.at[pl.ds(dbase, C)]], drows.at[0], sem))
        ops.append(pltpu.make_async_copy(
            outemb_hbm.at[outidx_v.at[pl.ds(dbase, C)]], drows.at[1], sem))
        return ops

    def fire(cc, rows, drows, sem):
        for op in gather_ops(cc, rows, drows, sem):
            op.start()

    def drain(cc, rows, drows, sem):
        for op in gather_ops(cc, rows, drows, sem):
            op.wait()

    def out_op(cc, obuf, sem):
        g0 = nbase + cc * C
        b = g0 // N
        n0 = g0 % N
        return pltpu.make_async_copy(obuf, out_hbm.at[pl.ds(1 + n0, C), b],
                                     sem)

    def compute_and_store(cc, rows, drows, obuf, semw, first):
        @pl.when(jnp.logical_not(first))
        def _():
            out_op(cc, obuf, semw).wait()

        @plsc.parallel_loop(0, C, 1, unroll=2)
        def node_body(n):
            r0 = n * F
            for k in range(H // 16):
                sl = pl.ds(k * 16, 16)
                acc = drows[0, n, sl] + drows[1, n, sl]
                for f in range(F):
                    acc = acc + rows[r0 + f, sl]
                obuf[n, sl] = acc

        out_op(cc, obuf, semw).start()

    fire(0, rows0, drows0, sem0)

    def loop_body(g, carry):
        c0 = g * 2
        c1 = c0 + 1
        fire(c1, rows1, drows1, sem1)
        drain(c0, rows0, drows0, sem0)
        compute_and_store(c0, rows0, drows0, obuf0, semw0, g == 0)

        @pl.when(g < CHUNKS // 2 - 1)
        def _():
            fire(c0 + 2, rows0, drows0, sem0)
        drain(c1, rows1, drows1, sem1)
        compute_and_store(c1, rows1, drows1, obuf1, semw1, g == 0)
        return carry

    lax.fori_loop(0, CHUNKS // 2, loop_body, 0)
    out_op(CHUNKS - 2, obuf0, semw0).wait()
    out_op(CHUNKS - 1, obuf1, semw1).wait()


@jax.jit
def _run(x_flat, ind_flat, outd_flat, atom_t, in_deg_emb, out_deg_emb,
         gt8):
    mesh = plsc.VectorSubcoreMesh(core_axis_name="c", subcore_axis_name="s")
    out = pl.kernel(
        _sc_body,
        out_type=jax.ShapeDtypeStruct((N + 1, B, H), jnp.float32),
        mesh=mesh,
        compiler_params=pltpu.CompilerParams(use_tc_tiling_on_sc=False),
        scratch_types=[
            pltpu.VMEM((XPW,), jnp.int32),
            pltpu.VMEM((NODES_PER_W,), jnp.int32),
            pltpu.VMEM((NODES_PER_W,), jnp.int32),
            pltpu.VMEM((RPC, H), jnp.float32),
            pltpu.VMEM((RPC, H), jnp.float32),
            pltpu.VMEM((2, C, H), jnp.float32),
            pltpu.VMEM((2, C, H), jnp.float32),
            pltpu.VMEM((C, H), jnp.float32),
            pltpu.VMEM((C, H), jnp.float32),
            pltpu.VMEM((2, H), jnp.float32),
            pltpu.SemaphoreType.DMA,
            pltpu.SemaphoreType.DMA,
            pltpu.SemaphoreType.DMA,
            pltpu.SemaphoreType.DMA,
        ],
    )(x_flat, ind_flat, outd_flat, atom_t, in_deg_emb, out_deg_emb, gt8)
    return out.transpose((1, 0, 2))


def kernel(x, in_degree, out_degree, atom_emb, in_deg_emb, out_deg_emb,
           graph_token):
    x_flat = x.astype(jnp.int32).reshape(-1)
    ind_flat = jnp.minimum(in_degree.astype(jnp.int32).reshape(-1), 511)
    outd_flat = jnp.minimum(out_degree.astype(jnp.int32).reshape(-1), 511)
    gt8 = jnp.tile(graph_token, (8, 1))
    return _run(x_flat, ind_flat, outd_flat, atom_emb, in_deg_emb,
                out_deg_emb, gt8)
